# baseline (speedup 1.0000x reference)
import functools

import jax
import jax.numpy as jnp
import numpy as np
from jax import lax
from jax.experimental import pallas as pl
from jax.experimental.pallas import tpu as pltpu
from jax.experimental.pallas import tpu_sc as plsc

N_ROWS = 2048
N_SEL = 1024
H = 14
W = 14
C = 96

NC = 2
NS = 16
NW = NC * NS

S_TOT = H * W * (C // 8)
NITER = 37

SLAB_IN = (N_ROWS // 128) * 8 * 128
SLAB_OUT = (N_SEL // 128) * 8 * 128
PAD = 16


def _build_plan(sel):
    rank = np.cumsum(sel) - sel
    plan = []
    for w in range(N_ROWS // 16):
        m = sel[w * 16:(w + 1) * 16]
        pc = int(m.sum())
        if pc == 0:
            continue
        src_off = (w // 8) * 1024 + (w % 8) * 16
        j0 = int(rank[w * 16 + int(np.argmax(m))])
        stores = []
        if j0 // 128 == (j0 + pc - 1) // 128:
            stores.append((m.copy(), (j0 // 128) * 1024 + (j0 % 128)))
        else:
            pc1 = 128 - (j0 % 128)
            m_a = m.copy()
            m_b = m.copy()
            cnt = 0
            for i in range(16):
                if m[i]:
                    if cnt < pc1:
                        m_b[i] = False
                    else:
                        m_a[i] = False
                    cnt += 1
            stores.append((m_a, (j0 // 128) * 1024 + (j0 % 128)))
            stores.append((m_b, ((j0 + pc1) // 128) * 1024))
        plan.append((src_off, stores))
    return plan


def _lane_select(x_flat, mask_tab, plan):
    mesh = plsc.VectorSubcoreMesh(core_axis_name="c", subcore_axis_name="s")
    n_mask_words = mask_tab.shape[0]

    @functools.partial(
        pl.kernel,
        mesh=mesh,
        compiler_params=pltpu.CompilerParams(needs_layout_passes=False),
        out_type=jax.ShapeDtypeStruct((S_TOT * SLAB_OUT,), jnp.float32),
        scratch_types=(
            pltpu.VMEM((SLAB_IN,), jnp.float32),
            pltpu.VMEM((SLAB_IN,), jnp.float32),
            pltpu.VMEM((SLAB_OUT + PAD,), jnp.float32),
            pltpu.VMEM((SLAB_OUT + PAD,), jnp.float32),
            pltpu.VMEM((n_mask_words,), jnp.int32),
            pltpu.SemaphoreType.DMA,
            pltpu.SemaphoreType.DMA,
            pltpu.SemaphoreType.DMA,
            pltpu.SemaphoreType.DMA,
        ),
    )
    def k(x_hbm, mask_hbm, out_hbm, in0, in1, out0, out1, mask_v,
          si0, si1, so0, so1):
        wid = lax.axis_index("s") * NC + lax.axis_index("c")
        pltpu.sync_copy(mask_hbm, mask_v)
        ins = (in0, in1)
        outs = (out0, out1)
        sis = (si0, si1)
        sos = (so0, so1)

        def slab(g):
            return g * NW + wid

        def start_in(g, b):
            @pl.when(slab(g) < S_TOT)
            def _():
                pltpu.async_copy(
                    x_hbm.at[pl.ds(slab(g) * SLAB_IN, SLAB_IN)], ins[b], sis[b])

        def wait_in(g, b):
            @pl.when(slab(g) < S_TOT)
            def _():
                pltpu.make_async_copy(
                    x_hbm.at[pl.ds(0, SLAB_IN)], ins[b], sis[b]).wait()

        def start_out(g, b):
            @pl.when(slab(g) < S_TOT)
            def _():
                pltpu.async_copy(
                    outs[b].at[pl.ds(0, SLAB_OUT)],
                    out_hbm.at[pl.ds(slab(g) * SLAB_OUT, SLAB_OUT)],
                    sos[b])

        def wait_out(g, b):
            @pl.when(jnp.logical_and(g >= 0, slab(g) < S_TOT))
            def _():
                pltpu.make_async_copy(
                    outs[b].at[pl.ds(0, SLAB_OUT)],
                    out_hbm.at[pl.ds(0, SLAB_OUT)], sos[b]).wait()

        def compute(b):
            src = ins[b]
            dst = outs[b]

            def crbody(cr, carry):
                crb = cr * 128
                for src_off, stores in plan:
                    v = src[pl.ds(src_off + crb, 16)]
                    for mi, dst_off in stores:
                        mv = mask_v[pl.ds(mi * 16, 16)] != 0
                        plsc.store_compressed(
                            dst.at[pl.ds(dst_off + crb, 16)], v, mask=mv)
                return carry

            lax.fori_loop(0, 8, crbody, 0)

        start_in(0, 0)
        start_in(1, 1)

        def body(g, carry):
            for b in range(2):
                gg = 2 * g + b
                wait_in(gg, b)
                wait_out(gg - 2, b)
                compute(b)
                start_out(gg, b)
                start_in(gg + 2, b)
            return carry

        lax.fori_loop(0, NITER, body, 0)
        for b in range(2):
            wait_out(2 * (NITER - 1) + b, b)

    return k(x_flat, mask_tab)


def kernel(x):
    with jax.ensure_compile_time_eval():
        ck = jax.random.key(42)
        choices = jax.random.choice(ck, N_ROWS, shape=(N_SEL,), replace=False)
        choices = np.asarray(jnp.sort(choices))
    sel = np.zeros(N_ROWS, dtype=bool)
    sel[choices] = True
    mask_list = []
    plan = []
    for src_off, stores in _build_plan(sel):
        idx_stores = []
        for m, dst_off in stores:
            idx_stores.append((len(mask_list), dst_off))
            mask_list.append(m.astype(np.int32))
        plan.append((src_off, idx_stores))
    mask_tab = jnp.asarray(np.concatenate(mask_list))
    x_flat = (
        x.reshape(16, 128, 12, 8, H, W)
        .transpose(4, 5, 2, 0, 3, 1)
        .reshape(S_TOT * SLAB_IN)
    )
    out_flat = _lane_select(x_flat, mask_tab, plan)
    out = (
        out_flat.reshape(H, W, 12, N_SEL // 128, 8, 128)
        .transpose(3, 5, 2, 4, 0, 1)
        .reshape(N_SEL, C, H, W)
    )
    return out


# speedup vs baseline: 5.3725x; 5.3725x over previous
import functools

import jax
import jax.numpy as jnp
from jax import lax
from jax.experimental import pallas as pl
from jax.experimental.pallas import tpu as pltpu
from jax.experimental.pallas import tpu_sc as plsc

N_ROWS = 2048
N_SEL = 1024
H = 14
W = 14
C = 96

NC = 2
NS = 16
NW = NC * NS

S_TOT = H * W * (C // 8)
BT_IN = N_ROWS // 128
BT_OUT = N_SEL // 128
NITER = 37
JV = N_SEL // 16


SLAB_IN = BT_IN * 8 * 128
SLAB_OUT = BT_OUT * 8 * 128


def _lane_gather(x_flat, base):
    mesh = plsc.VectorSubcoreMesh(core_axis_name="c", subcore_axis_name="s")

    @functools.partial(
        pl.kernel,
        mesh=mesh,
        compiler_params=pltpu.CompilerParams(needs_layout_passes=False),
        out_type=jax.ShapeDtypeStruct((S_TOT * SLAB_OUT,), jnp.float32),
        scratch_types=(
            pltpu.VMEM((N_SEL,), jnp.int32),
            pltpu.VMEM((SLAB_IN,), jnp.float32),
            pltpu.VMEM((SLAB_IN,), jnp.float32),
            pltpu.VMEM((SLAB_OUT,), jnp.float32),
            pltpu.VMEM((SLAB_OUT,), jnp.float32),
            pltpu.SemaphoreType.DMA,
            pltpu.SemaphoreType.DMA,
            pltpu.SemaphoreType.DMA,
            pltpu.SemaphoreType.DMA,
        ),
    )
    def k(x_hbm, base_hbm, out_hbm, base_v, in0, in1, out0, out1,
          si0, si1, so0, so1):
        wid = lax.axis_index("s") * NC + lax.axis_index("c")
        pltpu.sync_copy(base_hbm, base_v)
        ins = (in0, in1)
        outs = (out0, out1)
        sis = (si0, si1)
        sos = (so0, so1)

        def slab(g):
            return g * NW + wid

        def start_in(g, b):
            @pl.when(slab(g) < S_TOT)
            def _():
                pltpu.async_copy(
                    x_hbm.at[pl.ds(slab(g) * SLAB_IN, SLAB_IN)], ins[b], sis[b])

        def wait_in(g, b):
            @pl.when(slab(g) < S_TOT)
            def _():
                pltpu.make_async_copy(
                    x_hbm.at[pl.ds(0, SLAB_IN)], ins[b], sis[b]).wait()

        def start_out(g, b):
            @pl.when(slab(g) < S_TOT)
            def _():
                pltpu.async_copy(
                    outs[b], out_hbm.at[pl.ds(slab(g) * SLAB_OUT, SLAB_OUT)],
                    sos[b])

        def wait_out(g, b):
            @pl.when(jnp.logical_and(g >= 0, slab(g) < S_TOT))
            def _():
                pltpu.make_async_copy(
                    outs[b], out_hbm.at[pl.ds(0, SLAB_OUT)], sos[b]).wait()

        def compute(b):
            src = ins[b]
            dst = outs[b]

            @plsc.parallel_loop(0, JV, 1, unroll=8)
            def _(jv):
                bvec = base_v[pl.ds(jv * 16, 16)]
                o = (jv // 8) * 1024 + (jv % 8) * 16
                for cr in range(8):
                    v = plsc.load_gather(src, [bvec + cr * 128])
                    dst[pl.ds(o + cr * 128, 16)] = v

        start_in(0, 0)
        start_in(1, 1)

        def body(g, carry):
            for b in range(2):
                gg = 2 * g + b
                wait_in(gg, b)
                wait_out(gg - 2, b)
                compute(b)
                start_out(gg, b)
                start_in(gg + 2, b)
            return carry

        lax.fori_loop(0, NITER, body, 0)
        for b in range(2):
            wait_out(2 * (NITER - 1) + b, b)

    return k(x_flat, base)


def kernel(x):
    with jax.ensure_compile_time_eval():
        ck = jax.random.key(42)
        choices = jax.random.choice(ck, N_ROWS, shape=(N_SEL,), replace=False)
        choices = jnp.sort(choices).astype(jnp.int32)
        base = (choices + (choices >> 7) * 896).astype(jnp.int32)
    x_flat = (
        x.reshape(16, 128, 12, 8, H, W)
        .transpose(4, 5, 2, 0, 3, 1)
        .reshape(S_TOT * SLAB_IN)
    )
    out_flat = _lane_gather(x_flat, base)
    out = (
        out_flat.reshape(H, W, 12, BT_OUT, 8, 128)
        .transpose(3, 5, 2, 4, 0, 1)
        .reshape(N_SEL, C, H, W)
    )
    return out


# speedup vs baseline: 5.6239x; 1.0468x over previous
import functools

import jax
import jax.numpy as jnp
from jax import lax
from jax.experimental import pallas as pl
from jax.experimental.pallas import tpu as pltpu
from jax.experimental.pallas import tpu_sc as plsc

N_ROWS = 2048
N_SEL = 1024
H = 14
W = 14
C = 96

NC = 2
NS = 16
NW = NC * NS

S_TOT = H * W * (C // 8)
BT_IN = N_ROWS // 128
BT_OUT = N_SEL // 128
SPC = 2
C_TOT = S_TOT // SPC
NITER = 19
JV = N_SEL // 16


SLAB_IN = BT_IN * 8 * 128
SLAB_OUT = BT_OUT * 8 * 128


def _lane_gather(x_flat, base):
    mesh = plsc.VectorSubcoreMesh(core_axis_name="c", subcore_axis_name="s")

    @functools.partial(
        pl.kernel,
        mesh=mesh,
        compiler_params=pltpu.CompilerParams(needs_layout_passes=False),
        out_type=jax.ShapeDtypeStruct((S_TOT * SLAB_OUT,), jnp.float32),
        scratch_types=(
            pltpu.VMEM((N_SEL,), jnp.int32),
            pltpu.VMEM((SPC * SLAB_IN,), jnp.float32),
            pltpu.VMEM((SPC * SLAB_IN,), jnp.float32),
            pltpu.VMEM((SPC * SLAB_OUT,), jnp.float32),
            pltpu.VMEM((SPC * SLAB_OUT,), jnp.float32),
            pltpu.SemaphoreType.DMA,
            pltpu.SemaphoreType.DMA,
            pltpu.SemaphoreType.DMA,
            pltpu.SemaphoreType.DMA,
        ),
    )
    def k(x_hbm, base_hbm, out_hbm, base_v, in0, in1, out0, out1,
          si0, si1, so0, so1):
        wid = lax.axis_index("s") * NC + lax.axis_index("c")
        pltpu.sync_copy(base_hbm, base_v)
        ins = (in0, in1)
        outs = (out0, out1)
        sis = (si0, si1)
        sos = (so0, so1)

        def chunk(g):
            return g * NW + wid

        def start_in(g, b):
            @pl.when(chunk(g) < C_TOT)
            def _():
                pltpu.async_copy(
                    x_hbm.at[pl.ds(chunk(g) * (SPC * SLAB_IN), SPC * SLAB_IN)],
                    ins[b], sis[b])

        def wait_in(g, b):
            @pl.when(chunk(g) < C_TOT)
            def _():
                pltpu.make_async_copy(
                    x_hbm.at[pl.ds(0, SPC * SLAB_IN)], ins[b], sis[b]).wait()

        def start_out(g, b):
            @pl.when(chunk(g) < C_TOT)
            def _():
                pltpu.async_copy(
                    outs[b],
                    out_hbm.at[pl.ds(chunk(g) * (SPC * SLAB_OUT),
                                     SPC * SLAB_OUT)],
                    sos[b])

        def wait_out(g, b):
            @pl.when(jnp.logical_and(g >= 0, chunk(g) < C_TOT))
            def _():
                pltpu.make_async_copy(
                    outs[b], out_hbm.at[pl.ds(0, SPC * SLAB_OUT)],
                    sos[b]).wait()

        def compute(b):
            src = ins[b]
            dst = outs[b]

            @plsc.parallel_loop(0, SPC * JV, 1, unroll=8)
            def _(j):
                sl = j // JV
                jv = j % JV
                bvec = base_v[pl.ds(jv * 16, 16)]
                o = (sl * SLAB_OUT + (jv // 8) * 1024 + (jv % 8) * 16)
                si = sl * SLAB_IN
                for cr in range(8):
                    v = plsc.load_gather(src, [bvec + (si + cr * 128)])
                    dst[pl.ds(o + cr * 128, 16)] = v

        start_in(0, 0)
        start_in(1, 1)

        def body(g, carry):
            for b in range(2):
                gg = 2 * g + b
                wait_in(gg, b)
                wait_out(gg - 2, b)
                compute(b)
                start_out(gg, b)
                start_in(gg + 2, b)
            return carry

        lax.fori_loop(0, NITER, body, 0)
        for b in range(2):
            wait_out(2 * (NITER - 1) + b, b)

    return k(x_flat, base)


def kernel(x):
    with jax.ensure_compile_time_eval():
        ck = jax.random.key(42)
        choices = jax.random.choice(ck, N_ROWS, shape=(N_SEL,), replace=False)
        choices = jnp.sort(choices).astype(jnp.int32)
        base = (choices + (choices >> 7) * 896).astype(jnp.int32)
    x_flat = (
        x.reshape(16, 128, 12, 8, H, W)
        .transpose(4, 5, 2, 0, 3, 1)
        .reshape(S_TOT * SLAB_IN)
    )
    out_flat = _lane_gather(x_flat, base)
    out = (
        out_flat.reshape(H, W, 12, BT_OUT, 8, 128)
        .transpose(3, 5, 2, 4, 0, 1)
        .reshape(N_SEL, C, H, W)
    )
    return out
